# baseline (speedup 1.0000x reference)
import functools

import jax
import jax.numpy as jnp
from jax import lax
from jax.experimental import pallas as pl
from jax.experimental.pallas import tpu as pltpu
from jax.experimental.pallas import tpu_sc as plsc

NUM_IDX = 16384 * 50
DIM = 64

_info = plsc.get_sparse_core_info()
_NC, _NS = _info.num_cores, _info.num_subcores
NW = _NC * _NS
B_PER_W = NUM_IDX // NW
CHUNK = 128
K = 2
BLK = K * CHUNK
N_BLOCKS = B_PER_W // BLK
N_PAIRS = N_BLOCKS // 2

_mesh = plsc.VectorSubcoreMesh(core_axis_name="c", subcore_axis_name="s")


@functools.partial(
    pl.kernel,
    mesh=_mesh,
    out_type=jax.ShapeDtypeStruct((NUM_IDX, DIM), jnp.float32),
    scratch_types=[
        pltpu.VMEM((B_PER_W,), jnp.int32),
        pltpu.VMEM((BLK, 128), jnp.float32),
        pltpu.VMEM((BLK, 128), jnp.float32),
        pltpu.SemaphoreType.DMA,
        pltpu.SemaphoreType.DMA,
        pltpu.SemaphoreType.DMA,
    ],
    compiler_params=pltpu.CompilerParams(use_tc_tiling_on_sc=False),
)
def _sc_gather(idx_hbm, table_hbm, out_hbm, idx_all, rows0, rows1,
               gsem, ssem0, ssem1):
    wid = lax.axis_index("s") * _NC + lax.axis_index("c")
    base = wid * B_PER_W

    pltpu.sync_copy(idx_hbm.at[pl.ds(base, B_PER_W)], idx_all)

    def fire_gathers(g, rows):
        for j in range(K):
            pltpu.async_copy(
                table_hbm.at[idx_all.at[pl.ds(g * BLK + j * CHUNK, CHUNK)]],
                rows.at[pl.ds(j * CHUNK, CHUNK)],
                gsem,
            )

    def wait_gathers(rows):
        pltpu.make_async_copy(table_hbm.at[pl.ds(0, BLK)], rows, gsem).wait()

    def fire_store(g, rows, sem):
        pltpu.async_copy(rows.at[:, pl.ds(0, DIM)],
                         out_hbm.at[pl.ds(base + g * BLK, BLK)], sem)

    def wait_store(rows, sem):
        pltpu.make_async_copy(rows.at[:, pl.ds(0, DIM)],
                              out_hbm.at[pl.ds(base, BLK)], sem).wait()

    fire_gathers(0, rows0)

    def body(p, carry):
        g0 = 2 * p
        wait_gathers(rows0)
        fire_store(g0, rows0, ssem0)

        @pl.when(p > 0)
        def _():
            wait_store(rows1, ssem1)

        fire_gathers(g0 + 1, rows1)
        wait_gathers(rows1)
        fire_store(g0 + 1, rows1, ssem1)

        @pl.when(p < N_PAIRS - 1)
        def _():
            wait_store(rows0, ssem0)
            fire_gathers(g0 + 2, rows0)

        return carry

    lax.fori_loop(0, N_PAIRS, body, 0)
    wait_store(rows0, ssem0)
    wait_store(rows1, ssem1)


def kernel(indices, weight):
    flat = indices.reshape(-1).astype(jnp.int32)
    wp = jnp.pad(weight, ((0, 0), (0, 128 - DIM)))
    out = _sc_gather(flat, wp)
    return out.reshape(indices.shape + (weight.shape[1],))


# speedup vs baseline: 1.1069x; 1.1069x over previous
import functools

import jax
import jax.numpy as jnp
from jax import lax
from jax.experimental import pallas as pl
from jax.experimental.pallas import tpu as pltpu
from jax.experimental.pallas import tpu_sc as plsc

NUM_IDX = 16384 * 50
DIM = 64

_info = plsc.get_sparse_core_info()
_NC, _NS = _info.num_cores, _info.num_subcores
NW = _NC * _NS
B_PER_W = NUM_IDX // NW
CHUNK = 128
K = 5
BLK = K * CHUNK
N_BLOCKS = B_PER_W // BLK
N_PAIRS = N_BLOCKS // 2

_mesh = plsc.VectorSubcoreMesh(core_axis_name="c", subcore_axis_name="s")


@functools.partial(
    pl.kernel,
    mesh=_mesh,
    out_type=jax.ShapeDtypeStruct((NUM_IDX, DIM), jnp.float32),
    scratch_types=[
        pltpu.VMEM((B_PER_W,), jnp.int32),
        pltpu.VMEM((BLK, DIM), jnp.float32),
        pltpu.VMEM((BLK, DIM), jnp.float32),
        pltpu.SemaphoreType.DMA,
        pltpu.SemaphoreType.DMA,
        pltpu.SemaphoreType.DMA,
    ],
    compiler_params=pltpu.CompilerParams(use_tc_tiling_on_sc=False),
)
def _sc_gather(idx_hbm, table_hbm, out_hbm, idx_all, rows0, rows1,
               gsem, ssem0, ssem1):
    wid = lax.axis_index("s") * _NC + lax.axis_index("c")
    base = wid * B_PER_W

    pltpu.sync_copy(idx_hbm.at[pl.ds(base, B_PER_W)], idx_all)

    def fire_gathers(g, rows):
        for j in range(K):
            pltpu.async_copy(
                table_hbm.at[idx_all.at[pl.ds(g * BLK + j * CHUNK, CHUNK)]],
                rows.at[pl.ds(j * CHUNK, CHUNK)],
                gsem,
            )

    def wait_gathers(rows):
        pltpu.make_async_copy(out_hbm.at[pl.ds(base, BLK)], rows, gsem).wait()

    def fire_store(g, rows, sem):
        pltpu.async_copy(rows, out_hbm.at[pl.ds(base + g * BLK, BLK)], sem)

    def wait_store(rows, sem):
        pltpu.make_async_copy(rows, out_hbm.at[pl.ds(base, BLK)], sem).wait()

    fire_gathers(0, rows0)

    def body(p, carry):
        g0 = 2 * p
        wait_gathers(rows0)
        fire_store(g0, rows0, ssem0)

        @pl.when(p > 0)
        def _():
            wait_store(rows1, ssem1)

        fire_gathers(g0 + 1, rows1)
        wait_gathers(rows1)
        fire_store(g0 + 1, rows1, ssem1)

        @pl.when(p < N_PAIRS - 1)
        def _():
            wait_store(rows0, ssem0)
            fire_gathers(g0 + 2, rows0)

        return carry

    lax.fori_loop(0, N_PAIRS, body, 0)
    wait_store(rows0, ssem0)
    wait_store(rows1, ssem1)


def kernel(indices, weight):
    flat = indices.reshape(-1).astype(jnp.int32)
    out = _sc_gather(flat, weight)
    return out.reshape(indices.shape + (weight.shape[1],))
